# baseline (speedup 1.0000x reference)
import functools

import jax
import jax.numpy as jnp
from jax import lax
from jax.experimental import pallas as pl
from jax.experimental.pallas import tpu as pltpu
from jax.experimental.pallas import tpu_sc as plsc

B = 16384
D = 64
H = 128
W = 2 * D

_NC = 2
_NS = 16
_NW = _NC * _NS
_CHUNK = 128
_ROWS = B // _CHUNK
_RPW = _ROWS // _NW


def _sc_gather3(p_idx2, w_idx2, l_idx2, prompt_t, video_t):
    mesh = plsc.VectorSubcoreMesh(
        core_axis_name="c", subcore_axis_name="s",
        num_cores=_NC, num_subcores=_NS)

    out_t = jax.ShapeDtypeStruct((_ROWS, _CHUNK, W), jnp.float32)

    @functools.partial(
        pl.kernel,
        out_type=(out_t, out_t, out_t),
        mesh=mesh,
        scratch_types=[
            pltpu.VMEM((_RPW, _CHUNK), jnp.int32),
            pltpu.VMEM((_RPW, _CHUNK), jnp.int32),
            pltpu.VMEM((_RPW, _CHUNK), jnp.int32),
            pltpu.VMEM((2, _CHUNK, W), jnp.float32),
            pltpu.VMEM((2, _CHUNK, W), jnp.float32),
            pltpu.VMEM((2, _CHUNK, W), jnp.float32),
            pltpu.SemaphoreType.DMA,
            pltpu.SemaphoreType.DMA,
        ],
    )
    def gather_kernel(p_idx_hbm, w_idx_hbm, l_idx_hbm, pt_hbm, vt_hbm,
                      out_p, out_w, out_l,
                      pidx_v, widx_v, lidx_v, pbuf, wbuf, lbuf, gsem, osem):
        wid = lax.axis_index("s") * _NC + lax.axis_index("c")
        base = wid * _RPW
        pltpu.sync_copy(p_idx_hbm.at[pl.ds(base, _RPW)], pidx_v)
        pltpu.sync_copy(w_idx_hbm.at[pl.ds(base, _RPW)], widx_v)
        pltpu.sync_copy(l_idx_hbm.at[pl.ds(base, _RPW)], lidx_v)
        tabs = ((pt_hbm, pidx_v, pbuf, out_p),
                (vt_hbm, widx_v, wbuf, out_w),
                (vt_hbm, lidx_v, lbuf, out_l))
        for wave in range(_RPW // 2):
            gs = []
            for tab, idxv, buf, _ in tabs:
                for k in range(2):
                    j = 2 * wave + k
                    gs.append(pltpu.async_copy(
                        tab.at[idxv.at[j]], buf.at[k], gsem))
            for g in gs:
                g.wait()
            os_ = []
            for _, _, buf, out in tabs:
                for k in range(2):
                    j = 2 * wave + k
                    os_.append(pltpu.async_copy(
                        buf.at[k], out.at[base + j], osem))
            for o in os_:
                o.wait()

    return gather_kernel(p_idx2, w_idx2, l_idx2, prompt_t, video_t)


_BLK = 4096
_INV_SQRT2 = 0.7071067811865476


def _gelu(x):
    return 0.5 * x * (1.0 + lax.erf(x * _INV_SQRT2))


def _mlp_body(p_ref, vw_ref, vl_ref, pp_ref, wp_ref, lp_ref,
              w1a_ref, w1b_ref, b1_ref, w2_ref, b2_ref, w3_ref, b3_ref,
              rw_ref, rl_ref):
    def sel(full_ref, par_ref):
        full = full_ref[...]
        par = par_ref[...]
        return jnp.where(par == 1, full[:, D:], full[:, :D])

    p = sel(p_ref, pp_ref)
    pa = jnp.dot(p, w1a_ref[...], preferred_element_type=jnp.float32)
    b1 = b1_ref[...]
    w2 = w2_ref[...]
    b2 = b2_ref[...]
    w3 = w3_ref[...]
    b3 = b3_ref[0, 0]
    for v_ref, vp_ref, out_ref in ((vw_ref, wp_ref, rw_ref),
                                   (vl_ref, lp_ref, rl_ref)):
        v = sel(v_ref, vp_ref)
        h = pa + jnp.dot(v, w1b_ref[...],
                         preferred_element_type=jnp.float32) + b1
        h = _gelu(h)
        h = jnp.dot(h, w2, preferred_element_type=jnp.float32) + b2
        h = _gelu(h)
        out_ref[...] = jnp.sum(h * w3, axis=1) + b3


def _mlp_head(p, vw, vl, pp, wp, lp, W1, b1, W2, b2, W3, b3):
    w1a = W1[:D]
    w1b = W1[D:]
    b1r = b1.reshape(1, H)
    b2r = b2.reshape(1, H)
    w3r = W3.reshape(1, H)
    b3r = b3.reshape(1, 1)
    grid = (B // _BLK,)
    row_spec = pl.BlockSpec((_BLK, W), lambda i: (i, 0))
    par_spec = pl.BlockSpec((_BLK, 1), lambda i: (i, 0))
    full = lambda shape: pl.BlockSpec(shape, lambda i: (0,) * len(shape))
    return pl.pallas_call(
        _mlp_body,
        grid=grid,
        in_specs=[
            row_spec, row_spec, row_spec,
            par_spec, par_spec, par_spec,
            full((D, H)), full((D, H)), full((1, H)),
            full((H, H)), full((1, H)), full((1, H)), full((1, 1)),
        ],
        out_specs=[pl.BlockSpec((_BLK,), lambda i: (i,)),
                   pl.BlockSpec((_BLK,), lambda i: (i,))],
        out_shape=[jax.ShapeDtypeStruct((B,), jnp.float32),
                   jax.ShapeDtypeStruct((B,), jnp.float32)],
        compiler_params=pltpu.CompilerParams(
            dimension_semantics=("parallel",)),
    )(p, vw, vl, pp, wp, lp, w1a, w1b, b1r, W2, b2r, w3r, b3r)


def kernel(prompt_idx, preferred_idx, rejected_idx, video_emb, prompt_emb,
           W1, b1, W2, b2, W3, b3):
    n_v = video_emb.shape[0]
    n_p = prompt_emb.shape[0]
    video_t = video_emb.reshape(n_v // 2, W)
    prompt_t = prompt_emb.reshape(n_p // 2, W)
    p_idx2 = (prompt_idx >> 1).reshape(_ROWS, _CHUNK)
    w_idx2 = (preferred_idx >> 1).reshape(_ROWS, _CHUNK)
    l_idx2 = (rejected_idx >> 1).reshape(_ROWS, _CHUNK)
    pp = (prompt_idx & 1).reshape(B, 1)
    wp = (preferred_idx & 1).reshape(B, 1)
    lp = (rejected_idx & 1).reshape(B, 1)
    p3, vw3, vl3 = _sc_gather3(p_idx2, w_idx2, l_idx2, prompt_t, video_t)
    p = p3.reshape(B, W)
    vw = vw3.reshape(B, W)
    vl = vl3.reshape(B, W)
    r_w, r_l = _mlp_head(p, vw, vl, pp, wp, lp, W1, b1, W2, b2, W3, b3)
    return (r_w, r_l)


# speedup vs baseline: 1.2127x; 1.2127x over previous
import functools

import jax
import jax.numpy as jnp
from jax import lax
from jax.experimental import pallas as pl
from jax.experimental.pallas import tpu as pltpu
from jax.experimental.pallas import tpu_sc as plsc

B = 16384
D = 64
H = 128

_NC = 2
_NS = 16
_NW = _NC * _NS
_CHUNK = 128
_ROWS = B // _CHUNK
_RPW = _ROWS // _NW


def _sc_gather3(p_idx2, w_idx2, l_idx2, pv, vv):
    mesh = plsc.VectorSubcoreMesh(
        core_axis_name="c", subcore_axis_name="s",
        num_cores=_NC, num_subcores=_NS)

    out_t = jax.ShapeDtypeStruct((_ROWS, _CHUNK, H), jnp.float32)

    @functools.partial(
        pl.kernel,
        out_type=(out_t, out_t, out_t),
        mesh=mesh,
        scratch_types=[
            pltpu.VMEM((_RPW, _CHUNK), jnp.int32),
            pltpu.VMEM((_RPW, _CHUNK), jnp.int32),
            pltpu.VMEM((_RPW, _CHUNK), jnp.int32),
            pltpu.VMEM((2, _CHUNK, H), jnp.float32),
            pltpu.VMEM((2, _CHUNK, H), jnp.float32),
            pltpu.VMEM((2, _CHUNK, H), jnp.float32),
            pltpu.SemaphoreType.DMA,
            pltpu.SemaphoreType.DMA,
        ],
    )
    def gather_kernel(p_idx_hbm, w_idx_hbm, l_idx_hbm, pv_hbm, vv_hbm,
                      out_p, out_w, out_l,
                      pidx_v, widx_v, lidx_v, pbuf, wbuf, lbuf, gsem, osem):
        wid = lax.axis_index("s") * _NC + lax.axis_index("c")
        base = wid * _RPW
        pltpu.sync_copy(p_idx_hbm.at[pl.ds(base, _RPW)], pidx_v)
        pltpu.sync_copy(w_idx_hbm.at[pl.ds(base, _RPW)], widx_v)
        pltpu.sync_copy(l_idx_hbm.at[pl.ds(base, _RPW)], lidx_v)
        tabs = ((pv_hbm, pidx_v, pbuf, out_p),
                (vv_hbm, widx_v, wbuf, out_w),
                (vv_hbm, lidx_v, lbuf, out_l))
        for wave in range(_RPW // 2):
            gs = []
            for tab, idxv, buf, _ in tabs:
                for k in range(2):
                    j = 2 * wave + k
                    gs.append(pltpu.async_copy(
                        tab.at[idxv.at[j]], buf.at[k], gsem))
            for g in gs:
                g.wait()
            os_ = []
            for _, _, buf, out in tabs:
                for k in range(2):
                    j = 2 * wave + k
                    os_.append(pltpu.async_copy(
                        buf.at[k], out.at[base + j], osem))
            for o in os_:
                o.wait()

    return gather_kernel(p_idx2, w_idx2, l_idx2, pv, vv)


_FBLK = 2048
_BLK = 4096
_INV_SQRT2 = 0.7071067811865476


def _gelu(x):
    return 0.5 * x * (1.0 + lax.erf(x * _INV_SQRT2))


def _fold_body(xt_ref, w_ref, out_ref):
    out_ref[...] = lax.dot_general(
        xt_ref[...], w_ref[...], (((0,), (0,)), ((), ())),
        preferred_element_type=jnp.float32)


def _fold(table_t, w):
    n = table_t.shape[1]
    grid = (pl.cdiv(n, _FBLK),)
    return pl.pallas_call(
        _fold_body,
        grid=grid,
        in_specs=[pl.BlockSpec((D, _FBLK), lambda i: (0, i)),
                  pl.BlockSpec((D, H), lambda i: (0, 0))],
        out_specs=pl.BlockSpec((_FBLK, H), lambda i: (i, 0)),
        out_shape=jax.ShapeDtypeStruct((n, H), jnp.float32),
        compiler_params=pltpu.CompilerParams(
            dimension_semantics=("parallel",)),
    )(table_t, w)


def _tail_body(pf_ref, vw_ref, vl_ref, b1_ref, w2_ref, b2_ref,
               w3_ref, b3_ref, rw_ref, rl_ref):
    pa = pf_ref[...] + b1_ref[...]
    w2 = w2_ref[...]
    b2 = b2_ref[...]
    w3 = w3_ref[...]
    b3 = b3_ref[0, 0]
    for v_ref, out_ref in ((vw_ref, rw_ref), (vl_ref, rl_ref)):
        h = _gelu(pa + v_ref[...])
        h = jnp.dot(h, w2, preferred_element_type=jnp.float32) + b2
        h = _gelu(h)
        out_ref[...] = jnp.sum(h * w3, axis=1) + b3


def _mlp_tail(pf, vwf, vlf, b1, W2, b2, W3, b3):
    b1r = b1.reshape(1, H)
    b2r = b2.reshape(1, H)
    w3r = W3.reshape(1, H)
    b3r = b3.reshape(1, 1)
    grid = (B // _BLK,)
    row_spec = pl.BlockSpec((_BLK, H), lambda i: (i, 0))
    full = lambda shape: pl.BlockSpec(shape, lambda i: (0,) * len(shape))
    return pl.pallas_call(
        _tail_body,
        grid=grid,
        in_specs=[
            row_spec, row_spec, row_spec,
            full((1, H)), full((H, H)), full((1, H)), full((1, H)),
            full((1, 1)),
        ],
        out_specs=[pl.BlockSpec((_BLK,), lambda i: (i,)),
                   pl.BlockSpec((_BLK,), lambda i: (i,))],
        out_shape=[jax.ShapeDtypeStruct((B,), jnp.float32),
                   jax.ShapeDtypeStruct((B,), jnp.float32)],
        compiler_params=pltpu.CompilerParams(
            dimension_semantics=("parallel",)),
    )(pf, vwf, vlf, b1r, W2, b2r, w3r, b3r)


def kernel(prompt_idx, preferred_idx, rejected_idx, video_emb, prompt_emb,
           W1, b1, W2, b2, W3, b3):
    vv = _fold(video_emb.T, W1[D:])
    pv = _fold(prompt_emb.T, W1[:D])
    p_idx2 = prompt_idx.reshape(_ROWS, _CHUNK)
    w_idx2 = preferred_idx.reshape(_ROWS, _CHUNK)
    l_idx2 = rejected_idx.reshape(_ROWS, _CHUNK)
    p3, vw3, vl3 = _sc_gather3(p_idx2, w_idx2, l_idx2, pv, vv)
    pf = p3.reshape(B, H)
    vwf = vw3.reshape(B, H)
    vlf = vl3.reshape(B, H)
    r_w, r_l = _mlp_tail(pf, vwf, vlf, b1, W2, b2, W3, b3)
    return (r_w, r_l)


# speedup vs baseline: 2.0157x; 1.6621x over previous
import functools

import jax
import jax.numpy as jnp
from jax import lax
from jax.experimental import pallas as pl
from jax.experimental.pallas import tpu as pltpu
from jax.experimental.pallas import tpu_sc as plsc

B = 16384
D = 64
H = 128

_NC = 2
_NS = 16
_NW = _NC * _NS
_CHUNK = 128
_ROWS = B // _CHUNK
_RPW = _ROWS // _NW


def _sc_gather3(p_idx2, w_idx2, l_idx2, pv, vv):
    mesh = plsc.VectorSubcoreMesh(
        core_axis_name="c", subcore_axis_name="s",
        num_cores=_NC, num_subcores=_NS)

    out_t = jax.ShapeDtypeStruct((_ROWS, _CHUNK, H), jnp.float32)

    @functools.partial(
        pl.kernel,
        out_type=(out_t, out_t, out_t),
        mesh=mesh,
        scratch_types=[
            pltpu.VMEM((_RPW, _CHUNK), jnp.int32),
            pltpu.VMEM((_RPW, _CHUNK), jnp.int32),
            pltpu.VMEM((_RPW, _CHUNK), jnp.int32),
            pltpu.VMEM((2, _CHUNK, H), jnp.float32),
            pltpu.VMEM((2, _CHUNK, H), jnp.float32),
            pltpu.VMEM((2, _CHUNK, H), jnp.float32),
            pltpu.SemaphoreType.DMA,
            pltpu.SemaphoreType.DMA,
        ],
    )
    def gather_kernel(p_idx_hbm, w_idx_hbm, l_idx_hbm, pv_hbm, vv_hbm,
                      out_p, out_w, out_l,
                      pidx_v, widx_v, lidx_v, pbuf, wbuf, lbuf, gsem, osem):
        wid = lax.axis_index("s") * _NC + lax.axis_index("c")
        base = wid * _RPW
        pltpu.sync_copy(p_idx_hbm.at[pl.ds(base, _RPW)], pidx_v)
        pltpu.sync_copy(w_idx_hbm.at[pl.ds(base, _RPW)], widx_v)
        pltpu.sync_copy(l_idx_hbm.at[pl.ds(base, _RPW)], lidx_v)
        tabs = ((pv_hbm, pidx_v, pbuf, out_p),
                (vv_hbm, widx_v, wbuf, out_w),
                (vv_hbm, lidx_v, lbuf, out_l))
        for wave in range(_RPW // 2):
            gs = []
            for tab, idxv, buf, _ in tabs:
                for k in range(2):
                    j = 2 * wave + k
                    gs.append(pltpu.async_copy(
                        tab.at[idxv.at[j]], buf.at[k], gsem))
            for g in gs:
                g.wait()
            os_ = []
            for _, _, buf, out in tabs:
                for k in range(2):
                    j = 2 * wave + k
                    os_.append(pltpu.async_copy(
                        buf.at[k], out.at[base + j], osem))
            for o in os_:
                o.wait()

    return gather_kernel(p_idx2, w_idx2, l_idx2, pv, vv)


_FBLK = 8192
_BLK = 4096
_INV_SQRT2 = 0.7071067811865476


def _gelu(x):
    return 0.5 * x * (1.0 + lax.erf(x * _INV_SQRT2))


def _fold_body(xt_ref, w_ref, out_ref):
    out_ref[...] = lax.dot_general(
        xt_ref[...], w_ref[...], (((0,), (0,)), ((), ())),
        preferred_element_type=jnp.float32)


def _fold(table_t, w):
    n = table_t.shape[1]
    grid = (pl.cdiv(n, _FBLK),)
    return pl.pallas_call(
        _fold_body,
        grid=grid,
        in_specs=[pl.BlockSpec((D, _FBLK), lambda i: (0, i)),
                  pl.BlockSpec((D, H), lambda i: (0, 0))],
        out_specs=pl.BlockSpec((_FBLK, H), lambda i: (i, 0)),
        out_shape=jax.ShapeDtypeStruct((n, H), jnp.float32),
        compiler_params=pltpu.CompilerParams(
            dimension_semantics=("parallel",),
            fuse_transposed_lhs_in_matmul=True),
    )(table_t, w)


def _tail_body(pf_ref, vw_ref, vl_ref, b1_ref, w2_ref, b2_ref,
               w3_ref, b3_ref, rw_ref, rl_ref):
    pa = pf_ref[...] + b1_ref[...]
    w2 = w2_ref[...]
    b2 = b2_ref[...]
    w3 = w3_ref[...]
    b3 = b3_ref[0, 0]
    for v_ref, out_ref in ((vw_ref, rw_ref), (vl_ref, rl_ref)):
        h = _gelu(pa + v_ref[...])
        h = jnp.dot(h, w2, preferred_element_type=jnp.float32) + b2
        h = _gelu(h)
        out_ref[...] = jnp.sum(h * w3, axis=1) + b3


def _mlp_tail(pf, vwf, vlf, b1, W2, b2, W3, b3):
    b1r = b1.reshape(1, H)
    b2r = b2.reshape(1, H)
    w3r = W3.reshape(1, H)
    b3r = b3.reshape(1, 1)
    grid = (B // _BLK,)
    row_spec = pl.BlockSpec((_BLK, H), lambda i: (i, 0))
    full = lambda shape: pl.BlockSpec(shape, lambda i: (0,) * len(shape))
    return pl.pallas_call(
        _tail_body,
        grid=grid,
        in_specs=[
            row_spec, row_spec, row_spec,
            full((1, H)), full((H, H)), full((1, H)), full((1, H)),
            full((1, 1)),
        ],
        out_specs=[pl.BlockSpec((_BLK,), lambda i: (i,)),
                   pl.BlockSpec((_BLK,), lambda i: (i,))],
        out_shape=[jax.ShapeDtypeStruct((B,), jnp.float32),
                   jax.ShapeDtypeStruct((B,), jnp.float32)],
        compiler_params=pltpu.CompilerParams(
            dimension_semantics=("parallel",)),
    )(pf, vwf, vlf, b1r, W2, b2r, w3r, b3r)


def kernel(prompt_idx, preferred_idx, rejected_idx, video_emb, prompt_emb,
           W1, b1, W2, b2, W3, b3):
    vv = _fold(video_emb.T, W1[D:])
    pv = _fold(prompt_emb.T, W1[:D])
    p_idx2 = prompt_idx.reshape(_ROWS, _CHUNK)
    w_idx2 = preferred_idx.reshape(_ROWS, _CHUNK)
    l_idx2 = rejected_idx.reshape(_ROWS, _CHUNK)
    p3, vw3, vl3 = _sc_gather3(p_idx2, w_idx2, l_idx2, pv, vv)
    pf = p3.reshape(B, H)
    vwf = vw3.reshape(B, H)
    vlf = vl3.reshape(B, H)
    r_w, r_l = _mlp_tail(pf, vwf, vlf, b1, W2, b2, W3, b3)
    return (r_w, r_l)


# speedup vs baseline: 2.4271x; 1.2041x over previous
import functools

import jax
import jax.numpy as jnp
from jax import lax
from jax.experimental import pallas as pl
from jax.experimental.pallas import tpu as pltpu
from jax.experimental.pallas import tpu_sc as plsc

B = 16384
D = 64
H = 128

_NC = 2
_NS = 16
_NW = _NC * _NS
_CHUNK = 128
_ROWS = B // _CHUNK
_RPW = _ROWS // _NW

_FBLK = 8192
_NV = 1000000
_NP = 100000
_KV = (_NV // 2 // _FBLK) * _FBLK
_KP = (_NP // 2 // _FBLK) * _FBLK
_MV = _NV - _KV
_MP = _NP - _KP


def _sc_gather3(p_idx2, w_idx2, l_idx2, pv, vv):
    mesh = plsc.VectorSubcoreMesh(
        core_axis_name="c", subcore_axis_name="s",
        num_cores=_NC, num_subcores=_NS)

    out_t = jax.ShapeDtypeStruct((_ROWS, _CHUNK, H), jnp.uint32)

    @functools.partial(
        pl.kernel,
        out_type=(out_t, out_t, out_t),
        mesh=mesh,
        scratch_types=[
            pltpu.VMEM((_RPW, _CHUNK), jnp.int32),
            pltpu.VMEM((_RPW, _CHUNK), jnp.int32),
            pltpu.VMEM((_RPW, _CHUNK), jnp.int32),
            pltpu.VMEM((2, _CHUNK, H), jnp.uint32),
            pltpu.VMEM((2, _CHUNK, H), jnp.uint32),
            pltpu.VMEM((2, _CHUNK, H), jnp.uint32),
            pltpu.SemaphoreType.DMA,
            pltpu.SemaphoreType.DMA,
        ],
    )
    def gather_kernel(p_idx_hbm, w_idx_hbm, l_idx_hbm, pv_hbm, vv_hbm,
                      out_p, out_w, out_l,
                      pidx_v, widx_v, lidx_v, pbuf, wbuf, lbuf, gsem, osem):
        wid = lax.axis_index("s") * _NC + lax.axis_index("c")
        base = wid * _RPW
        pltpu.sync_copy(p_idx_hbm.at[pl.ds(base, _RPW)], pidx_v)
        pltpu.sync_copy(w_idx_hbm.at[pl.ds(base, _RPW)], widx_v)
        pltpu.sync_copy(l_idx_hbm.at[pl.ds(base, _RPW)], lidx_v)
        tabs = ((pv_hbm, pidx_v, pbuf, out_p),
                (vv_hbm, widx_v, wbuf, out_w),
                (vv_hbm, lidx_v, lbuf, out_l))
        for wave in range(_RPW // 2):
            gs = []
            for tab, idxv, buf, _ in tabs:
                for k in range(2):
                    j = 2 * wave + k
                    gs.append(pltpu.async_copy(
                        tab.at[idxv.at[j]], buf.at[k], gsem))
            for g in gs:
                g.wait()
            os_ = []
            for _, _, buf, out in tabs:
                for k in range(2):
                    j = 2 * wave + k
                    os_.append(pltpu.async_copy(
                        buf.at[k], out.at[base + j], osem))
            for o in os_:
                o.wait()

    return gather_kernel(p_idx2, w_idx2, l_idx2, pv, vv)


_BLK = 4096
_INV_SQRT2 = 0.7071067811865476


def _gelu(x):
    return 0.5 * x * (1.0 + lax.erf(x * _INV_SQRT2))


def _tdot(xt, w):
    return lax.dot_general(xt, w, (((0,), (0,)), ((), ())),
                           preferred_element_type=jnp.float32)


def _fold_body(xlo_ref, xhi_ref, w_ref, out_ref):
    w = w_ref[...]
    lo = _tdot(xlo_ref[...], w).astype(jnp.bfloat16).astype(jnp.float32)
    hi = _tdot(xhi_ref[...], w).astype(jnp.bfloat16).astype(jnp.float32)
    lob = lax.bitcast_convert_type(lo, jnp.uint32) >> 16
    hib = lax.bitcast_convert_type(hi, jnp.uint32) & jnp.uint32(0xFFFF0000)
    out_ref[...] = lob | hib


def _fold(table_t, w, k_split, m):
    kb = k_split // _FBLK
    grid = (pl.cdiv(m, _FBLK),)
    return pl.pallas_call(
        _fold_body,
        grid=grid,
        in_specs=[pl.BlockSpec((D, _FBLK), lambda i: (0, i)),
                  pl.BlockSpec((D, _FBLK), lambda i: (0, i + kb)),
                  pl.BlockSpec((D, H), lambda i: (0, 0))],
        out_specs=pl.BlockSpec((_FBLK, H), lambda i: (i, 0)),
        out_shape=jax.ShapeDtypeStruct((m, H), jnp.uint32),
        compiler_params=pltpu.CompilerParams(
            dimension_semantics=("parallel",),
            fuse_transposed_lhs_in_matmul=True),
    )(table_t, table_t, w)


def _tail_body(pf_ref, vw_ref, vl_ref, ph_ref, wh_ref, lh_ref,
               b1_ref, w2_ref, b2_ref, w3_ref, b3_ref, rw_ref, rl_ref):
    def unpack(x_ref, h_ref):
        x = x_ref[...]
        hi_sel = h_ref[...]
        lo = lax.bitcast_convert_type(x << 16, jnp.float32)
        hi = lax.bitcast_convert_type(x & jnp.uint32(0xFFFF0000),
                                      jnp.float32)
        return jnp.where(hi_sel == 1, hi, lo)

    pa = unpack(pf_ref, ph_ref) + b1_ref[...]
    w2 = w2_ref[...]
    b2 = b2_ref[...]
    w3 = w3_ref[...]
    b3 = b3_ref[0, 0]
    for v_ref, h_ref, out_ref in ((vw_ref, wh_ref, rw_ref),
                                  (vl_ref, lh_ref, rl_ref)):
        h = _gelu(pa + unpack(v_ref, h_ref))
        h = jnp.dot(h, w2, preferred_element_type=jnp.float32) + b2
        h = _gelu(h)
        out_ref[...] = jnp.sum(h * w3, axis=1) + b3


def _mlp_tail(pf, vwf, vlf, ph, wh, lh, b1, W2, b2, W3, b3):
    b1r = b1.reshape(1, H)
    b2r = b2.reshape(1, H)
    w3r = W3.reshape(1, H)
    b3r = b3.reshape(1, 1)
    grid = (B // _BLK,)
    row_spec = pl.BlockSpec((_BLK, H), lambda i: (i, 0))
    h_spec = pl.BlockSpec((_BLK, 1), lambda i: (i, 0))
    full = lambda shape: pl.BlockSpec(shape, lambda i: (0,) * len(shape))
    return pl.pallas_call(
        _tail_body,
        grid=grid,
        in_specs=[
            row_spec, row_spec, row_spec,
            h_spec, h_spec, h_spec,
            full((1, H)), full((H, H)), full((1, H)), full((1, H)),
            full((1, 1)),
        ],
        out_specs=[pl.BlockSpec((_BLK,), lambda i: (i,)),
                   pl.BlockSpec((_BLK,), lambda i: (i,))],
        out_shape=[jax.ShapeDtypeStruct((B,), jnp.float32),
                   jax.ShapeDtypeStruct((B,), jnp.float32)],
        compiler_params=pltpu.CompilerParams(
            dimension_semantics=("parallel",)),
    )(pf, vwf, vlf, ph, wh, lh, b1r, W2, b2r, w3r, b3r)


def _split_idx(idx, k_split):
    hi = (idx >= k_split).astype(jnp.int32)
    g = idx - hi * k_split
    return g.reshape(_ROWS, _CHUNK), hi.reshape(B, 1)


def kernel(prompt_idx, preferred_idx, rejected_idx, video_emb, prompt_emb,
           W1, b1, W2, b2, W3, b3):
    vv = _fold(video_emb.T, W1[D:], _KV, _MV)
    pv = _fold(prompt_emb.T, W1[:D], _KP, _MP)
    p_idx2, ph = _split_idx(prompt_idx, _KP)
    w_idx2, wh = _split_idx(preferred_idx, _KV)
    l_idx2, lh = _split_idx(rejected_idx, _KV)
    p3, vw3, vl3 = _sc_gather3(p_idx2, w_idx2, l_idx2, pv, vv)
    pf = p3.reshape(B, H)
    vwf = vw3.reshape(B, H)
    vlf = vl3.reshape(B, H)
    r_w, r_l = _mlp_tail(pf, vwf, vlf, ph, wh, lh, b1, W2, b2, W3, b3)
    return (r_w, r_l)


# speedup vs baseline: 2.4356x; 1.0035x over previous
import functools

import jax
import jax.numpy as jnp
from jax import lax
from jax.experimental import pallas as pl
from jax.experimental.pallas import tpu as pltpu
from jax.experimental.pallas import tpu_sc as plsc

B = 16384
D = 64
H = 128

_NC = 2
_NS = 16
_NW = _NC * _NS
_CHUNK = 128
_ROWS = B // _CHUNK
_RPW = _ROWS // _NW

_FBLK = 8192
_NV = 1000000
_NP = 100000
_KV = (_NV // 2 // _FBLK) * _FBLK
_KP = (_NP // 2 // _FBLK) * _FBLK
_MV = _NV - _KV
_MP = _NP - _KP


def _sc_gather3(p_idx2, w_idx2, l_idx2, pv, vv):
    mesh = plsc.VectorSubcoreMesh(
        core_axis_name="c", subcore_axis_name="s",
        num_cores=_NC, num_subcores=_NS)

    out_t = jax.ShapeDtypeStruct((_ROWS, _CHUNK, H), jnp.uint32)

    @functools.partial(
        pl.kernel,
        out_type=(out_t, out_t, out_t),
        mesh=mesh,
        scratch_types=[
            pltpu.VMEM((_RPW, _CHUNK), jnp.int32),
            pltpu.VMEM((_RPW, _CHUNK), jnp.int32),
            pltpu.VMEM((_RPW, _CHUNK), jnp.int32),
            pltpu.VMEM((2, _CHUNK, H), jnp.uint32),
            pltpu.VMEM((2, _CHUNK, H), jnp.uint32),
            pltpu.VMEM((2, _CHUNK, H), jnp.uint32),
            pltpu.SemaphoreType.DMA,
            pltpu.SemaphoreType.DMA,
        ],
    )
    def gather_kernel(p_idx_hbm, w_idx_hbm, l_idx_hbm, pv_hbm, vv_hbm,
                      out_p, out_w, out_l,
                      pidx_v, widx_v, lidx_v, pbuf, wbuf, lbuf, gsem, osem):
        wid = lax.axis_index("s") * _NC + lax.axis_index("c")
        base = wid * _RPW
        pltpu.sync_copy(p_idx_hbm.at[pl.ds(base, _RPW)], pidx_v)
        pltpu.sync_copy(w_idx_hbm.at[pl.ds(base, _RPW)], widx_v)
        pltpu.sync_copy(l_idx_hbm.at[pl.ds(base, _RPW)], lidx_v)
        tabs = ((pv_hbm, pidx_v, pbuf, out_p),
                (vv_hbm, widx_v, wbuf, out_w),
                (vv_hbm, lidx_v, lbuf, out_l))
        for wave in range(_RPW // 2):
            gs = []
            for tab, idxv, buf, _ in tabs:
                for k in range(2):
                    j = 2 * wave + k
                    gs.append(pltpu.async_copy(
                        tab.at[idxv.at[j]], buf.at[k], gsem))
            for g in gs:
                g.wait()
            os_ = []
            for _, _, buf, out in tabs:
                for k in range(2):
                    j = 2 * wave + k
                    os_.append(pltpu.async_copy(
                        buf.at[k], out.at[base + j], osem))
            for o in os_:
                o.wait()

    return gather_kernel(p_idx2, w_idx2, l_idx2, pv, vv)


_BLK = 4096
_INV_SQRT2 = 0.7071067811865476


def _gelu(x):
    return 0.5 * x * (1.0 + lax.erf(x * _INV_SQRT2))


def _tdot(xt, w):
    return lax.dot_general(xt, w, (((0,), (0,)), ((), ())),
                           preferred_element_type=jnp.float32)


_GV = -(-_MV // _FBLK)
_GP = -(-_MP // _FBLK)


def _pack(xlo, xhi, w):
    lo = _tdot(xlo, w).astype(jnp.bfloat16).astype(jnp.float32)
    hi = _tdot(xhi, w).astype(jnp.bfloat16).astype(jnp.float32)
    lob = lax.bitcast_convert_type(lo, jnp.uint32) >> 16
    hib = lax.bitcast_convert_type(hi, jnp.uint32) & jnp.uint32(0xFFFF0000)
    return lob | hib


def _fold_body(vlo_ref, vhi_ref, plo_ref, phi_ref, wb_ref, wa_ref,
               vv_ref, pv_ref):
    step = pl.program_id(0)

    @pl.when(step < _GV)
    def _():
        vv_ref[...] = _pack(vlo_ref[...], vhi_ref[...], wb_ref[...])

    @pl.when(step >= _GV)
    def _():
        pv_ref[...] = _pack(plo_ref[...], phi_ref[...], wa_ref[...])


def _fold2(video_t, prompt_t, w1b, w1a):
    kv = _KV // _FBLK
    kp = _KP // _FBLK
    vb = lambda i: jnp.minimum(i, _GV - 1)
    pb = lambda i: jnp.clip(i - _GV, 0, _GP - 1)
    return pl.pallas_call(
        _fold_body,
        grid=(_GV + _GP,),
        in_specs=[pl.BlockSpec((D, _FBLK), lambda i: (0, vb(i))),
                  pl.BlockSpec((D, _FBLK), lambda i: (0, vb(i) + kv)),
                  pl.BlockSpec((D, _FBLK), lambda i: (0, pb(i))),
                  pl.BlockSpec((D, _FBLK), lambda i: (0, pb(i) + kp)),
                  pl.BlockSpec((D, H), lambda i: (0, 0)),
                  pl.BlockSpec((D, H), lambda i: (0, 0))],
        out_specs=[pl.BlockSpec((_FBLK, H), lambda i: (vb(i), 0)),
                   pl.BlockSpec((_FBLK, H), lambda i: (pb(i), 0))],
        out_shape=[jax.ShapeDtypeStruct((_MV, H), jnp.uint32),
                   jax.ShapeDtypeStruct((_MP, H), jnp.uint32)],
        compiler_params=pltpu.CompilerParams(
            dimension_semantics=("arbitrary",),
            fuse_transposed_lhs_in_matmul=True),
    )(video_t, video_t, prompt_t, prompt_t, w1b, w1a)


def _tail_body(pf_ref, vw_ref, vl_ref, ph_ref, wh_ref, lh_ref,
               b1_ref, w2_ref, b2_ref, w3_ref, b3_ref, rw_ref, rl_ref):
    def unpack(x_ref, h_ref):
        x = x_ref[...]
        hi_sel = h_ref[...]
        lo = lax.bitcast_convert_type(x << 16, jnp.float32)
        hi = lax.bitcast_convert_type(x & jnp.uint32(0xFFFF0000),
                                      jnp.float32)
        return jnp.where(hi_sel == 1, hi, lo)

    pa = unpack(pf_ref, ph_ref) + b1_ref[...]
    w2 = w2_ref[...]
    b2 = b2_ref[...]
    w3 = w3_ref[...]
    b3 = b3_ref[0, 0]
    for v_ref, h_ref, out_ref in ((vw_ref, wh_ref, rw_ref),
                                  (vl_ref, lh_ref, rl_ref)):
        h = _gelu(pa + unpack(v_ref, h_ref))
        h = jnp.dot(h, w2, preferred_element_type=jnp.float32) + b2
        h = _gelu(h)
        out_ref[...] = jnp.sum(h * w3, axis=1) + b3


def _mlp_tail(pf, vwf, vlf, ph, wh, lh, b1, W2, b2, W3, b3):
    b1r = b1.reshape(1, H)
    b2r = b2.reshape(1, H)
    w3r = W3.reshape(1, H)
    b3r = b3.reshape(1, 1)
    grid = (B // _BLK,)
    row_spec = pl.BlockSpec((_BLK, H), lambda i: (i, 0))
    h_spec = pl.BlockSpec((_BLK, 1), lambda i: (i, 0))
    full = lambda shape: pl.BlockSpec(shape, lambda i: (0,) * len(shape))
    return pl.pallas_call(
        _tail_body,
        grid=grid,
        in_specs=[
            row_spec, row_spec, row_spec,
            h_spec, h_spec, h_spec,
            full((1, H)), full((H, H)), full((1, H)), full((1, H)),
            full((1, 1)),
        ],
        out_specs=[pl.BlockSpec((_BLK,), lambda i: (i,)),
                   pl.BlockSpec((_BLK,), lambda i: (i,))],
        out_shape=[jax.ShapeDtypeStruct((B,), jnp.float32),
                   jax.ShapeDtypeStruct((B,), jnp.float32)],
        compiler_params=pltpu.CompilerParams(
            dimension_semantics=("parallel",)),
    )(pf, vwf, vlf, ph, wh, lh, b1r, W2, b2r, w3r, b3r)


def _split_idx(idx, k_split):
    hi = (idx >= k_split).astype(jnp.int32)
    g = idx - hi * k_split
    return g.reshape(_ROWS, _CHUNK), hi.reshape(B, 1)


def kernel(prompt_idx, preferred_idx, rejected_idx, video_emb, prompt_emb,
           W1, b1, W2, b2, W3, b3):
    vv, pv = _fold2(video_emb.T, prompt_emb.T, W1[D:], W1[:D])
    p_idx2, ph = _split_idx(prompt_idx, _KP)
    w_idx2, wh = _split_idx(preferred_idx, _KV)
    l_idx2, lh = _split_idx(rejected_idx, _KV)
    p3, vw3, vl3 = _sc_gather3(p_idx2, w_idx2, l_idx2, pv, vv)
    pf = p3.reshape(B, H)
    vwf = vw3.reshape(B, H)
    vlf = vl3.reshape(B, H)
    r_w, r_l = _mlp_tail(pf, vwf, vlf, ph, wh, lh, b1, W2, b2, W3, b3)
    return (r_w, r_l)


# speedup vs baseline: 2.6966x; 1.1072x over previous
import functools

import jax
import jax.numpy as jnp
from jax import lax
from jax.experimental import pallas as pl
from jax.experimental.pallas import tpu as pltpu
from jax.experimental.pallas import tpu_sc as plsc

B = 16384
D = 64
H = 128

_NC = 2
_NS = 16
_NW = _NC * _NS
_CHUNK = 128
_ROWS = B // _CHUNK
_RPW = _ROWS // _NW

_FBLK = 8192
_NV = 1000000
_NP = 100000
_KV = (_NV // 2 // _FBLK) * _FBLK
_KP = (_NP // 2 // _FBLK) * _FBLK
_MV = _NV - _KV
_MP = _NP - _KP


def _sc_gather3(p_idx2, w_idx2, l_idx2, pv, vv):
    mesh = plsc.VectorSubcoreMesh(
        core_axis_name="c", subcore_axis_name="s",
        num_cores=_NC, num_subcores=_NS)

    out_t = jax.ShapeDtypeStruct((_ROWS, _CHUNK, H), jnp.uint32)

    @functools.partial(
        pl.kernel,
        out_type=(out_t, out_t, out_t),
        mesh=mesh,
        scratch_types=[
            pltpu.VMEM((_RPW, _CHUNK), jnp.int32),
            pltpu.VMEM((_RPW, _CHUNK), jnp.int32),
            pltpu.VMEM((_RPW, _CHUNK), jnp.int32),
            pltpu.VMEM((2, _CHUNK, H), jnp.uint32),
            pltpu.VMEM((2, _CHUNK, H), jnp.uint32),
            pltpu.VMEM((2, _CHUNK, H), jnp.uint32),
            pltpu.SemaphoreType.DMA,
            pltpu.SemaphoreType.DMA,
        ],
    )
    def gather_kernel(p_idx_hbm, w_idx_hbm, l_idx_hbm, pv_hbm, vv_hbm,
                      out_p, out_w, out_l,
                      pidx_v, widx_v, lidx_v, pbuf, wbuf, lbuf, gsem, osem):
        wid = lax.axis_index("s") * _NC + lax.axis_index("c")
        base = wid * _RPW
        pltpu.sync_copy(p_idx_hbm.at[pl.ds(base, _RPW)], pidx_v)
        pltpu.sync_copy(w_idx_hbm.at[pl.ds(base, _RPW)], widx_v)
        pltpu.sync_copy(l_idx_hbm.at[pl.ds(base, _RPW)], lidx_v)
        tabs = ((pv_hbm, pidx_v, pbuf, out_p),
                (vv_hbm, widx_v, wbuf, out_w),
                (vv_hbm, lidx_v, lbuf, out_l))
        for wave in range(_RPW // 2):
            gs = []
            for tab, idxv, buf, _ in tabs:
                for k in range(2):
                    j = 2 * wave + k
                    gs.append(pltpu.async_copy(
                        tab.at[idxv.at[j]], buf.at[k], gsem))
            for g in gs:
                g.wait()
            os_ = []
            for _, _, buf, out in tabs:
                for k in range(2):
                    j = 2 * wave + k
                    os_.append(pltpu.async_copy(
                        buf.at[k], out.at[base + j], osem))
            for o in os_:
                o.wait()

    return gather_kernel(p_idx2, w_idx2, l_idx2, pv, vv)


_BLK = 4096
_INV_SQRT2 = 0.7071067811865476


def _gelu(x):
    return 0.5 * x * (1.0 + lax.erf(x * _INV_SQRT2))


def _tdot(xt, w):
    return lax.dot_general(xt, w, (((0,), (0,)), ((), ())),
                           preferred_element_type=jnp.float32)


_GV = -(-_MV // _FBLK)
_GP = -(-_MP // _FBLK)


def _pack(xlo, xhi, w):
    w16 = w.astype(jnp.bfloat16)
    lo = _tdot(xlo.astype(jnp.bfloat16),
               w16).astype(jnp.bfloat16).astype(jnp.float32)
    hi = _tdot(xhi.astype(jnp.bfloat16),
               w16).astype(jnp.bfloat16).astype(jnp.float32)
    lob = lax.bitcast_convert_type(lo, jnp.uint32) >> 16
    hib = lax.bitcast_convert_type(hi, jnp.uint32) & jnp.uint32(0xFFFF0000)
    return lob | hib


def _fold_body(vlo_ref, vhi_ref, plo_ref, phi_ref, wb_ref, wa_ref,
               vv_ref, pv_ref):
    step = pl.program_id(0)

    @pl.when(step < _GV)
    def _():
        vv_ref[...] = _pack(vlo_ref[...], vhi_ref[...], wb_ref[...])

    @pl.when(step >= _GV)
    def _():
        pv_ref[...] = _pack(plo_ref[...], phi_ref[...], wa_ref[...])


def _fold2(video_t, prompt_t, w1b, w1a):
    kv = _KV // _FBLK
    kp = _KP // _FBLK
    vb = lambda i: jnp.minimum(i, _GV - 1)
    pb = lambda i: jnp.clip(i - _GV, 0, _GP - 1)
    return pl.pallas_call(
        _fold_body,
        grid=(_GV + _GP,),
        in_specs=[pl.BlockSpec((D, _FBLK), lambda i: (0, vb(i))),
                  pl.BlockSpec((D, _FBLK), lambda i: (0, vb(i) + kv)),
                  pl.BlockSpec((D, _FBLK), lambda i: (0, pb(i))),
                  pl.BlockSpec((D, _FBLK), lambda i: (0, pb(i) + kp)),
                  pl.BlockSpec((D, H), lambda i: (0, 0)),
                  pl.BlockSpec((D, H), lambda i: (0, 0))],
        out_specs=[pl.BlockSpec((_FBLK, H), lambda i: (vb(i), 0)),
                   pl.BlockSpec((_FBLK, H), lambda i: (pb(i), 0))],
        out_shape=[jax.ShapeDtypeStruct((_MV, H), jnp.uint32),
                   jax.ShapeDtypeStruct((_MP, H), jnp.uint32)],
        compiler_params=pltpu.CompilerParams(
            dimension_semantics=("arbitrary",),
            fuse_transposed_lhs_in_matmul=True),
    )(video_t, video_t, prompt_t, prompt_t, w1b, w1a)


def _tail_body(pf_ref, vw_ref, vl_ref, ph_ref, wh_ref, lh_ref,
               b1_ref, w2_ref, b2_ref, w3_ref, b3_ref, rw_ref, rl_ref):
    def unpack(x_ref, h_ref):
        x = x_ref[...]
        hi_sel = h_ref[...]
        lo = lax.bitcast_convert_type(x << 16, jnp.float32)
        hi = lax.bitcast_convert_type(x & jnp.uint32(0xFFFF0000),
                                      jnp.float32)
        return jnp.where(hi_sel == 1, hi, lo)

    pa = unpack(pf_ref, ph_ref) + b1_ref[...]
    w2 = w2_ref[...]
    b2 = b2_ref[...]
    w3 = w3_ref[...]
    b3 = b3_ref[0, 0]
    for v_ref, h_ref, out_ref in ((vw_ref, wh_ref, rw_ref),
                                  (vl_ref, lh_ref, rl_ref)):
        h = _gelu(pa + unpack(v_ref, h_ref))
        h = jnp.dot(h, w2, preferred_element_type=jnp.float32) + b2
        h = _gelu(h)
        out_ref[...] = jnp.sum(h * w3, axis=1) + b3


def _mlp_tail(pf, vwf, vlf, ph, wh, lh, b1, W2, b2, W3, b3):
    b1r = b1.reshape(1, H)
    b2r = b2.reshape(1, H)
    w3r = W3.reshape(1, H)
    b3r = b3.reshape(1, 1)
    grid = (B // _BLK,)
    row_spec = pl.BlockSpec((_BLK, H), lambda i: (i, 0))
    h_spec = pl.BlockSpec((_BLK, 1), lambda i: (i, 0))
    full = lambda shape: pl.BlockSpec(shape, lambda i: (0,) * len(shape))
    return pl.pallas_call(
        _tail_body,
        grid=grid,
        in_specs=[
            row_spec, row_spec, row_spec,
            h_spec, h_spec, h_spec,
            full((1, H)), full((H, H)), full((1, H)), full((1, H)),
            full((1, 1)),
        ],
        out_specs=[pl.BlockSpec((_BLK,), lambda i: (i,)),
                   pl.BlockSpec((_BLK,), lambda i: (i,))],
        out_shape=[jax.ShapeDtypeStruct((B,), jnp.float32),
                   jax.ShapeDtypeStruct((B,), jnp.float32)],
        compiler_params=pltpu.CompilerParams(
            dimension_semantics=("parallel",)),
    )(pf, vwf, vlf, ph, wh, lh, b1r, W2, b2r, w3r, b3r)


def _split_idx(idx, k_split):
    hi = (idx >= k_split).astype(jnp.int32)
    g = idx - hi * k_split
    return g.reshape(_ROWS, _CHUNK), hi.reshape(B, 1)


def kernel(prompt_idx, preferred_idx, rejected_idx, video_emb, prompt_emb,
           W1, b1, W2, b2, W3, b3):
    vv, pv = _fold2(video_emb.T, prompt_emb.T, W1[D:], W1[:D])
    p_idx2, ph = _split_idx(prompt_idx, _KP)
    w_idx2, wh = _split_idx(preferred_idx, _KV)
    l_idx2, lh = _split_idx(rejected_idx, _KV)
    p3, vw3, vl3 = _sc_gather3(p_idx2, w_idx2, l_idx2, pv, vv)
    pf = p3.reshape(B, H)
    vwf = vw3.reshape(B, H)
    vlf = vl3.reshape(B, H)
    r_w, r_l = _mlp_tail(pf, vwf, vlf, ph, wh, lh, b1, W2, b2, W3, b3)
    return (r_w, r_l)
